# baseline (speedup 1.0000x reference)
import functools

import jax
import jax.numpy as jnp
from jax import lax
from jax.experimental import pallas as pl
from jax.experimental.pallas import tpu as pltpu
from jax.experimental.pallas import tpu_sc as plsc

NUM_EXPERTS = 8
TOP_K = 2
SL = 2048
HS = 768
FF = 3072
M = SL * TOP_K

TM = 512
NT = M // TM
L = NT + NUM_EXPERTS - 1
FFT = 768
F = FF // FFT

TMS = 512
NTS = SL // TMS


def _grouped_body(meta_ref, x_ref, w1_ref, w2_ref, scale_ref, y_ref):
    l = pl.program_id(0)
    f = pl.program_id(1)
    first = meta_ref[64 + l]

    @pl.when((f == 0) & (first == 1))
    def _():
        y_ref[...] = jnp.zeros_like(y_ref)

    h = jax.nn.gelu(jnp.dot(x_ref[...].astype(jnp.bfloat16),
                            w1_ref[0].astype(jnp.bfloat16),
                            preferred_element_type=jnp.float32))
    m = meta_ref[16 + l]
    start = meta_ref[32 + l]
    end = meta_ref[48 + l]
    rows = m * TM + jax.lax.broadcasted_iota(jnp.int32, (TM, 1), 0)
    scale = scale_ref[0, 0, :].reshape(TM, 1)
    scale = jnp.where((rows >= start) & (rows < end), scale, 0.0)
    y_ref[...] += jnp.dot((h * scale).astype(jnp.bfloat16),
                          w2_ref[0].astype(jnp.bfloat16),
                          preferred_element_type=jnp.float32)


def _grouped_gemm(xs, w1, w2, scale_tiles, meta, interpret=False):
    grid_spec = pltpu.PrefetchScalarGridSpec(
        num_scalar_prefetch=1,
        grid=(L, F),
        in_specs=[
            pl.BlockSpec((TM, HS), lambda l, f, meta: (meta[16 + l], 0)),
            pl.BlockSpec((1, HS, FFT), lambda l, f, meta: (meta[l], 0, f)),
            pl.BlockSpec((1, FFT, HS), lambda l, f, meta: (meta[l], f, 0)),
            pl.BlockSpec((1, 1, TM), lambda l, f, meta: (meta[16 + l], 0, 0)),
        ],
        out_specs=pl.BlockSpec((TM, HS), lambda l, f, meta: (meta[16 + l], 0)),
    )
    return pl.pallas_call(
        _grouped_body,
        grid_spec=grid_spec,
        out_shape=jax.ShapeDtypeStruct((M, HS), jnp.float32),
        interpret=interpret,
    )(meta, xs, w1, w2, scale_tiles)


def _shared_body(x_ref, w1_ref, w2_ref, c0_ref, c1_ref, y_ref):
    f = pl.program_id(1)

    @pl.when(f == 0)
    def _():
        y_ref[...] = jnp.zeros_like(y_ref)

    h = jax.nn.gelu(jnp.dot(x_ref[...].astype(jnp.bfloat16),
                            w1_ref[...].astype(jnp.bfloat16),
                            preferred_element_type=jnp.float32))
    y_ref[...] += jnp.dot(h.astype(jnp.bfloat16),
                          w2_ref[...].astype(jnp.bfloat16),
                          preferred_element_type=jnp.float32)

    @pl.when(f == F - 1)
    def _():
        y_ref[...] += c0_ref[...] + c1_ref[...]


def _shared_mlp(xf, w1_s, w2_s, c, interpret=False):
    return pl.pallas_call(
        _shared_body,
        grid=(NTS, F),
        in_specs=[
            pl.BlockSpec((TMS, HS), lambda m, f: (m, 0)),
            pl.BlockSpec((HS, FFT), lambda m, f: (0, f)),
            pl.BlockSpec((FFT, HS), lambda m, f: (f, 0)),
            pl.BlockSpec((TMS, HS), lambda m, f: (m, 0)),
            pl.BlockSpec((TMS, HS), lambda m, f: (m + NTS, 0)),
        ],
        out_specs=pl.BlockSpec((TMS, HS), lambda m, f: (m, 0)),
        out_shape=jax.ShapeDtypeStruct((SL, HS), jnp.float32),
        interpret=interpret,
    )(xf, w1_s, w2_s, c, c)


NW = 32
SPW = M // NW
_SHIFT = TM.bit_length() - 1


def _route_body(top_hbm, ewt_hbm, xf_hbm, xs_hbm, pos_hbm, ws_hbm, meta_hbm,
                top_v, rows_v, posn_v, wval_v, meta_v, sem):
    cid = lax.axis_index("c")
    sid = lax.axis_index("s")
    wid = sid * 2 + cid
    base = wid * SPW
    tbase = (wid % (NW // TOP_K)) * SPW

    pltpu.sync_copy(top_hbm, top_v)

    lane = lax.iota(jnp.int32, 16)
    zeros = jnp.zeros((16,), jnp.int32)
    cut = wid * (SPW // 16)

    def hist_step(v, carry):
        a0, a1, a2, a3, s0, s1, s2, s3 = carry
        snap = jnp.broadcast_to(v == cut, (16,))
        s0 = jnp.where(snap, a0, s0)
        s1 = jnp.where(snap, a1, s1)
        s2 = jnp.where(snap, a2, s2)
        s3 = jnp.where(snap, a3, s3)
        keys = top_v[pl.ds(v * 16, 16)]
        val = jnp.left_shift(1, jnp.left_shift(keys & 1, 4))
        pair = lax.shift_right_logical(keys, 1)
        a0 = a0 + jnp.where(pair == 0, val, 0)
        a1 = a1 + jnp.where(pair == 1, val, 0)
        a2 = a2 + jnp.where(pair == 2, val, 0)
        a3 = a3 + jnp.where(pair == 3, val, 0)
        return a0, a1, a2, a3, s0, s1, s2, s3

    acc = lax.fori_loop(0, M // 16, hist_step, (zeros,) * 8, unroll=8)

    hist = zeros
    pre = zeros
    for e in range(NUM_EXPERTS):
        sh = (e & 1) * 16
        he = jnp.sum(lax.shift_right_logical(acc[e >> 1], sh) & 0xFFFF)
        pe = jnp.sum(lax.shift_right_logical(acc[4 + (e >> 1)], sh) & 0xFFFF)
        hist = hist + jnp.where(lane == e, he, 0)
        pre = pre + jnp.where(lane == e, pe, 0)

    cum_incl = jnp.cumsum(hist)
    base_e = cum_incl - hist
    cur = base_e + pre

    for v in range(SPW // 16):
        keys = top_v[pl.ds(base + v * 16, 16)]
        pos = zeros
        add = zeros
        for e in range(NUM_EXPERTS):
            mvec = keys == e
            ranks = jnp.cumsum(jnp.where(mvec, 1, 0).astype(jnp.int32))
            cur_e = jnp.sum(jnp.where(lane == e, cur, 0))
            pos = jnp.where(mvec, cur_e + ranks - 1, pos)
            pc = plsc.all_reduce_population_count(mvec)
            add = add + jnp.where(lane == e, pc, 0)
        cur = cur + add
        posn_v[pl.ds(v * 16, 16)] = pos

    pltpu.sync_copy(posn_v, pos_hbm.at[pl.ds(base, SPW)])
    pltpu.sync_copy(ewt_hbm.at[pl.ds(base, SPW)], wval_v)
    pltpu.async_copy(wval_v, ws_hbm.at[posn_v], sem).wait()
    pltpu.sync_copy(xf_hbm.at[pl.ds(tbase, SPW)], rows_v)
    pltpu.async_copy(rows_v, xs_hbm.at[posn_v], sem).wait()

    @pl.when(wid == 0)
    def _():
        tpe = hist
        bins = cum_incl
        starts = base_e
        t1 = lax.shift_right_arithmetic(bins - 1, _SHIFT)
        t0 = lax.shift_right_arithmetic(starts, _SHIFT)
        n = jnp.where(tpe > 0, t1 - t0 + 1, 0)
        cumn = jnp.cumsum(n)

        l = lax.iota(jnp.int32, 16)

        def tiles(lv):
            g = jnp.zeros((16,), jnp.int32)
            sg = jnp.zeros((16,), jnp.int32)
            bg = jnp.zeros((16,), jnp.int32)
            jg = jnp.zeros((16,), jnp.int32)
            for e in range(NUM_EXPERTS):
                ce = jnp.sum(jnp.where(lane == e, cumn, 0))
                g = g + jnp.where(lv < ce, 0, 1)
            gc = jnp.minimum(g, NUM_EXPERTS - 1)
            for e in range(NUM_EXPERTS):
                sel = gc == e
                sg = jnp.where(sel, jnp.sum(jnp.where(lane == e, starts, 0)),
                               sg)
                bg = jnp.where(sel, jnp.sum(jnp.where(lane == e, bins, 0)),
                               bg)
                base_t = jnp.sum(jnp.where(lane == e, cumn - n, 0))
                jg = jnp.where(sel, lv - base_t, jg)
            m = lax.shift_right_arithmetic(sg, _SHIFT) + jg
            return gc, sg, bg, m

        total = jnp.sum(jnp.where(lane == NUM_EXPERTS - 1, cumn, 0))
        gc, sg, bg, m_raw = tiles(l)
        _, _, _, m_prev_raw = tiles(jnp.maximum(l - 1, 0))
        valid = l < total
        m_l = jnp.where(valid, m_raw, NT - 1)
        s_l = jnp.where(valid, sg, 1)
        e_l = jnp.where(valid, bg, 0)
        first = ((l == 0) | (m_raw != m_prev_raw)).astype(jnp.int32)
        first = jnp.where(valid, first, 0)
        meta_v[pl.ds(0, 16)] = gc
        meta_v[pl.ds(16, 16)] = m_l
        meta_v[pl.ds(32, 16)] = s_l
        meta_v[pl.ds(48, 16)] = e_l
        meta_v[pl.ds(64, 16)] = first
        pltpu.sync_copy(meta_v, meta_hbm)


def _route(top, ewt, xf):
    mesh = plsc.VectorSubcoreMesh(core_axis_name="c", subcore_axis_name="s", num_cores=2, num_subcores=16)
    f = functools.partial(
        pl.kernel,
        out_type=(
            jax.ShapeDtypeStruct((M, HS), jnp.float32),
            jax.ShapeDtypeStruct((M,), jnp.int32),
            jax.ShapeDtypeStruct((M,), jnp.float32),
            jax.ShapeDtypeStruct((80,), jnp.int32),
        ),
        mesh=mesh,
        compiler_params=pltpu.CompilerParams(needs_layout_passes=False),
        scratch_types=[
            pltpu.VMEM((M,), jnp.int32),
            pltpu.VMEM((SPW, HS), jnp.float32),
            pltpu.VMEM((SPW,), jnp.int32),
            pltpu.VMEM((SPW,), jnp.float32),
            pltpu.VMEM((80,), jnp.int32),
            pltpu.SemaphoreType.DMA,
        ],
    )(_route_body)
    return f(top, ewt, xf)


def _gather_body(y_hbm, pos_hbm, c_hbm, idx_v, rows_v, sem):
    cid = lax.axis_index("c")
    sid = lax.axis_index("s")
    wid = sid * 2 + cid
    base = wid * SPW
    pltpu.sync_copy(pos_hbm.at[pl.ds(base, SPW)], idx_v)
    pltpu.async_copy(y_hbm.at[idx_v], rows_v, sem).wait()
    pltpu.sync_copy(rows_v, c_hbm.at[pl.ds(base, SPW)])


def _unsort_gather(y, pos):
    mesh = plsc.VectorSubcoreMesh(core_axis_name="c", subcore_axis_name="s", num_cores=2, num_subcores=16)
    f = functools.partial(
        pl.kernel,
        out_type=jax.ShapeDtypeStruct((M, HS), jnp.float32),
        mesh=mesh,
        compiler_params=pltpu.CompilerParams(needs_layout_passes=False),
        scratch_types=[
            pltpu.VMEM((SPW,), jnp.int32),
            pltpu.VMEM((SPW, HS), jnp.float32),
            pltpu.SemaphoreType.DMA,
        ],
    )(_gather_body)
    return f(y, pos)


def kernel(x, expert_weights, expert_indices, w1, w2, w1_s, w2_s):
    in_shape = x.shape
    xf = x.reshape(-1, HS)

    top = expert_indices.reshape(SL, TOP_K).T.reshape(-1).astype(jnp.int32)
    ewt = expert_weights.reshape(SL, TOP_K).T.reshape(-1)

    xs, pos, ws, meta = _route(top, ewt, xf)

    scale_tiles = ws.reshape(NT, 1, TM)
    y = _grouped_gemm(xs, w1, w2, scale_tiles, meta)
    c = _unsort_gather(y, pos)
    out = _shared_mlp(xf, w1_s, w2_s, c)
    return out.reshape(in_shape)
